# baseline (speedup 1.0000x reference)
import jax
import jax.numpy as jnp
from jax import lax
from jax.experimental import pallas as pl
from jax.experimental.pallas import tpu as pltpu
from jax.experimental.pallas import tpu_sc as plsc

B = 16384
F = 26
V = 100000
D = 32

NC, NS, L = 2, 16, 16
NW = NC * NS

HM = 50048
CHB = 4096
NCHB = B // CHB
GRP = 8


def _kernel_body(idx_hbm, tab_hbm, out_hbm, row_a, row_b,
                 idx0, idx1, out_v, rsa, rsb, isem0, isem1, osem):
    d = lax.axis_index("s") * NC + lax.axis_index("c")
    idx_v = (idx0, idx1)
    isem = (isem0, isem1)
    lanes = lax.broadcasted_iota(jnp.int32, (L,), 0)

    pltpu.async_copy(tab_hbm.at[0, d, pl.ds(0, HM)], row_a, rsa)

    def field(f, _):
        pltpu.async_copy(tab_hbm.at[f, d, pl.ds(HM, V - HM)], row_b, rsb)
        pltpu.async_copy(idx_hbm.at[f, pl.ds(0, CHB)], idx_v[0], isem[0])
        pltpu.make_async_copy(tab_hbm.at[f, d, pl.ds(0, HM)], row_a, rsa).wait()
        @pl.when(f > 0)
        def _():
            pltpu.make_async_copy(out_v, out_hbm.at[0, 0], osem).wait()

        for h in range(NCHB):
            p = h % 2
            if h + 1 < NCHB:
                pltpu.async_copy(idx_hbm.at[f, pl.ds((h + 1) * CHB, CHB)],
                                 idx_v[1 - p], isem[1 - p])
            pltpu.make_async_copy(idx_hbm.at[f, pl.ds(0, CHB)], idx_v[p],
                                  isem[p]).wait()

            def chunk_a(i, _):
                base = i * (GRP * L)
                ids = [idx_v[p][pl.ds(base + k * L, L)] for k in range(GRP)]
                ms = [ids[k] < HM for k in range(GRP)]
                gs = [plsc.load_gather(row_a, [ids[k]], mask=ms[k])
                      for k in range(GRP)]
                for k in range(GRP):
                    out_v[pl.ds(h * CHB + base + k * L, L)] = gs[k]
                return 0

            lax.fori_loop(0, CHB // (GRP * L), chunk_a, 0, unroll=2)

        pltpu.make_async_copy(tab_hbm.at[f, d, pl.ds(HM, V - HM)], row_b, rsb).wait()

        @pl.when(f + 1 < F)
        def _():
            pltpu.async_copy(tab_hbm.at[f + 1, d, pl.ds(0, HM)], row_a, rsa)

        pltpu.async_copy(idx_hbm.at[f, pl.ds(0, CHB)], idx_v[0], isem[0])
        for h in range(NCHB):
            p = h % 2
            if h + 1 < NCHB:
                pltpu.async_copy(idx_hbm.at[f, pl.ds((h + 1) * CHB, CHB)],
                                 idx_v[1 - p], isem[1 - p])
            pltpu.make_async_copy(idx_hbm.at[f, pl.ds(0, CHB)], idx_v[p],
                                  isem[p]).wait()

            def chunk_b(i, _):
                base = i * (GRP * L)
                ids = [idx_v[p][pl.ds(base + k * L, L)] for k in range(GRP)]
                ms = [ids[k] >= HM for k in range(GRP)]
                gs = [plsc.load_gather(row_b, [ids[k] - HM], mask=ms[k])
                      for k in range(GRP)]
                for k in range(GRP):
                    pos = lanes + (h * CHB + base + k * L)
                    plsc.store_scatter(out_v, [pos], gs[k], mask=ms[k])
                return 0

            lax.fori_loop(0, CHB // (GRP * L), chunk_b, 0, unroll=2)

        pltpu.async_copy(out_v, out_hbm.at[f, d], osem)
        return 0

    lax.fori_loop(0, F, field, 0, unroll=False)
    pltpu.make_async_copy(out_v, out_hbm.at[0, 0], osem).wait()


@jax.jit
def _embed(idx_t, tab_t):
    mesh = plsc.VectorSubcoreMesh(
        core_axis_name="c", subcore_axis_name="s", num_cores=NC, num_subcores=NS
    )
    scratch = (
        [pltpu.VMEM((HM,), jnp.float32),
         pltpu.VMEM((V - HM,), jnp.float32)]
        + [pltpu.VMEM((CHB,), jnp.int32) for _ in range(2)]
        + [pltpu.VMEM((B,), jnp.float32)]
        + [pltpu.SemaphoreType.DMA for _ in range(5)]
    )
    return pl.kernel(
        _kernel_body,
        out_type=jax.ShapeDtypeStruct((F, D, B), jnp.float32),
        mesh=mesh,
        scratch_types=scratch,
        compiler_params=pltpu.CompilerParams(
            use_tc_tiling_on_sc=True, needs_layout_passes=False
        ),
    )(idx_t, tab_t)


def kernel(input, tables):
    idx_t = input.astype(jnp.int32).T
    tab_t = jnp.transpose(tables, (0, 2, 1))
    out_t = _embed(idx_t, tab_t)
    return jnp.transpose(out_t, (2, 0, 1))


# speedup vs baseline: 1.2548x; 1.2548x over previous
import jax
import jax.numpy as jnp
from jax import lax
from jax.experimental import pallas as pl
from jax.experimental.pallas import tpu as pltpu
from jax.experimental.pallas import tpu_sc as plsc

B = 16384
F = 26
V = 100000
D = 32

NC, NS, L = 2, 16, 16
NW = NC * NS


CHB = 4096
NCHB = B // CHB


def _kernel_body(idx_hbm, tab_hbm, out_hbm, row_v,
                 idx0, idx1, out0, out1, rsem, isem0, isem1, osem0, osem1):
    d = lax.axis_index("s") * NC + lax.axis_index("c")
    idx_v = (idx0, idx1)
    out_v = (out0, out1)
    isem = (isem0, isem1)
    osem = (osem0, osem1)

    def field(f, _):
        fb = f * B
        pltpu.async_copy(tab_hbm.at[f, d], row_v, rsem)
        pltpu.async_copy(idx_hbm.at[pl.ds(fb, CHB)], idx_v[0], isem[0])
        pltpu.make_async_copy(tab_hbm.at[f, d], row_v, rsem).wait()
        for h in range(NCHB):
            p = h % 2
            if h + 1 < NCHB:
                pltpu.async_copy(
                    idx_hbm.at[pl.ds(fb + (h + 1) * CHB, CHB)], idx_v[1 - p], isem[1 - p]
                )
            pltpu.make_async_copy(
                idx_hbm.at[pl.ds(fb, CHB)], idx_v[p], isem[p]
            ).wait()
            if h >= 2:
                pltpu.make_async_copy(out_v[p], out_hbm.at[0, 0, pl.ds(0, CHB)],
                                      osem[p]).wait()
            else:
                @pl.when(f > 0)
                def _():
                    pltpu.make_async_copy(out_v[p], out_hbm.at[0, 0, pl.ds(0, CHB)],
                                          osem[p]).wait()

            GRP = 8

            def chunk(i, _):
                base = i * (GRP * L)
                ids = [idx_v[p][pl.ds(base + k * L, L)] for k in range(GRP)]
                gs = [plsc.load_gather(row_v, [ids[k]]) for k in range(GRP)]
                for k in range(GRP):
                    out_v[p][pl.ds(base + k * L, L)] = gs[k]
                return 0

            lax.fori_loop(0, CHB // (GRP * L), chunk, 0, unroll=2)
            pltpu.async_copy(out_v[p], out_hbm.at[f, d, pl.ds(h * CHB, CHB)], osem[p])
        return 0

    lax.fori_loop(0, F, field, 0, unroll=False)
    for p in range(2):
        pltpu.make_async_copy(out_v[p], out_hbm.at[0, 0, pl.ds(0, CHB)], osem[p]).wait()


@jax.jit
def _embed(idx_t, tab_t):
    mesh = plsc.VectorSubcoreMesh(
        core_axis_name="c", subcore_axis_name="s", num_cores=NC, num_subcores=NS
    )
    scratch = (
        [pltpu.VMEM((V,), jnp.float32)]
        + [pltpu.VMEM((CHB,), jnp.int32) for _ in range(2)]
        + [pltpu.VMEM((CHB,), jnp.float32) for _ in range(2)]
        + [pltpu.SemaphoreType.DMA for _ in range(5)]
    )
    return pl.kernel(
        _kernel_body,
        out_type=jax.ShapeDtypeStruct((F, D, B), jnp.float32),
        mesh=mesh,
        scratch_types=scratch,
        compiler_params=pltpu.CompilerParams(
            use_tc_tiling_on_sc=True, needs_layout_passes=False
        ),
    )(idx_t, tab_t)


def kernel(input, tables):
    idx_f = input.astype(jnp.int32).T.reshape(F * B)
    tab_t = jnp.transpose(tables, (0, 2, 1))
    out_t = _embed(idx_f, tab_t)
    return jnp.transpose(out_t, (2, 0, 1))


# speedup vs baseline: 1.4145x; 1.1272x over previous
import jax
import jax.numpy as jnp
from jax import lax
from jax.experimental import pallas as pl
from jax.experimental.pallas import tpu as pltpu
from jax.experimental.pallas import tpu_sc as plsc

B = 16384
F = 26
V = 100000
D = 32

NC, NS, L = 2, 16, 16
NW = NC * NS


CHB = 4096
NCHB = B // CHB


def _kernel_body(idx_hbm, tab_hbm, out_hbm, row_v, idx_l, sidx,
                 out0, out1, rsem, ssem, lsem, osem0, osem1):
    c = lax.axis_index("c")
    s = lax.axis_index("s")
    d = s * NC + c
    out_v = (out0, out1)
    osem = (osem0, osem1)

    @pl.when(s == 0)
    def _():
        pltpu.async_copy(idx_hbm.at[pl.ds(0, B)], sidx.at[0], ssem)

    def field(f, _):
        @pl.when(s == 0)
        def _():
            pltpu.make_async_copy(idx_hbm.at[pl.ds(0, B)], sidx.at[0], ssem).wait()
        plsc.subcore_barrier()

        @pl.when((s == 0) & (f + 1 < F))
        def _():
            pltpu.async_copy(idx_hbm.at[pl.ds((f + 1) * B, B)],
                             sidx.at[lax.rem(f + 1, 2)], ssem)

        pltpu.async_copy(sidx.at[lax.rem(f, 2)], idx_l, lsem)
        pltpu.async_copy(tab_hbm.at[f, d], row_v, rsem)
        pltpu.make_async_copy(sidx.at[0], idx_l, lsem).wait()
        pltpu.make_async_copy(tab_hbm.at[f, d], row_v, rsem).wait()

        for h in range(NCHB):
            p = h % 2
            if h >= 2:
                pltpu.make_async_copy(out_v[p], out_hbm.at[0, 0, pl.ds(0, CHB)],
                                      osem[p]).wait()
            else:
                @pl.when(f > 0)
                def _():
                    pltpu.make_async_copy(out_v[p], out_hbm.at[0, 0, pl.ds(0, CHB)],
                                          osem[p]).wait()

            GRP = 8

            def chunk(i, _):
                base = i * (GRP * L)
                ids = [idx_l[pl.ds(h * CHB + base + k * L, L)] for k in range(GRP)]
                gs = [plsc.load_gather(row_v, [ids[k]]) for k in range(GRP)]
                for k in range(GRP):
                    out_v[p][pl.ds(base + k * L, L)] = gs[k]
                return 0

            lax.fori_loop(0, CHB // (GRP * L), chunk, 0, unroll=2)
            pltpu.async_copy(out_v[p], out_hbm.at[f, d, pl.ds(h * CHB, CHB)], osem[p])
        return 0

    lax.fori_loop(0, F, field, 0, unroll=False)
    for p in range(2):
        pltpu.make_async_copy(out_v[p], out_hbm.at[0, 0, pl.ds(0, CHB)], osem[p]).wait()


@jax.jit
def _embed(idx_t, tab_t):
    mesh = plsc.VectorSubcoreMesh(
        core_axis_name="c", subcore_axis_name="s", num_cores=NC, num_subcores=NS
    )
    scratch = (
        [pltpu.VMEM((V,), jnp.float32),
         pltpu.VMEM((B,), jnp.int32),
         pltpu.VMEM_SHARED((2, B), jnp.int32)]
        + [pltpu.VMEM((CHB,), jnp.float32) for _ in range(2)]
        + [pltpu.SemaphoreType.DMA for _ in range(5)]
    )
    return pl.kernel(
        _kernel_body,
        out_type=jax.ShapeDtypeStruct((F, D, B), jnp.float32),
        mesh=mesh,
        scratch_types=scratch,
        compiler_params=pltpu.CompilerParams(
            use_tc_tiling_on_sc=True, needs_layout_passes=False
        ),
    )(idx_t, tab_t)


def kernel(input, tables):
    idx_f = input.astype(jnp.int32).T.reshape(F * B)
    tab_t = jnp.transpose(tables, (0, 2, 1))
    out_t = _embed(idx_f, tab_t)
    return jnp.transpose(out_t, (2, 0, 1))
